# baseline (speedup 1.0000x reference)
import jax
import jax.numpy as jnp
from jax.experimental import pallas as pl
from jax.experimental.pallas import tpu as pltpu

_SIGMA = 5
_TEMP = 0.1
_MARGIN = 10.0
_ROWS = 2048
_STRIPS = 2
_CHUNK = 128


def _ce(a, b):
    return jnp.maximum(a, b), jnp.minimum(a, b)


def _merge22(x, y):
    p, q = _ce(x[0], y[0])
    r, s = _ce(x[1], y[1])
    m1, m2 = _ce(q, r)
    return [p, m1, m2, s]


def _oem44(a, b):
    e = _merge22([a[0], a[2]], [b[0], b[2]])
    o = _merge22([a[1], a[3]], [b[1], b[3]])
    h1, l1 = _ce(o[0], e[1])
    h2, l2 = _ce(o[1], e[2])
    h3, l3 = _ce(o[2], e[3])
    return [e[0], h1, l1, h2, l2, h3, l3, o[3]]


def _clean6(t, desc):
    t = list(t)
    for i in range(3):
        hi, lo = _ce(t[i], t[i + 3])
        t[i], t[i + 3] = (hi, lo) if desc else (lo, hi)
    for base in (0, 3):
        for (x, y) in ((0, 1), (1, 2), (0, 1)):
            hi, lo = _ce(t[base + x], t[base + y])
            t[base + x], t[base + y] = (hi, lo) if desc else (lo, hi)
    return t


def _merge_top6(a, b):
    t = [jnp.maximum(a[i], b[5 - i]) for i in range(6)]
    return _clean6(t, desc=True)


def _merge_bot6(a, b):
    t = [jnp.minimum(a[i], b[5 - i]) for i in range(6)]
    return _clean6(t, desc=False)


def _first_lane_mask(eq, lane, big):
    first = jnp.min(jnp.where(eq, lane, big), axis=1, keepdims=True)
    return lane == first


def _strip_v(sim):
    n_chunks = sim.shape[1] // _CHUNK
    chunks = [sim[:, c * _CHUNK:(c + 1) * _CHUNK].astype(jnp.bfloat16)
              for c in range(n_chunks)]

    l2 = []
    for j in range(n_chunks // 2):
        hi, lo = _ce(chunks[2 * j], chunks[2 * j + 1])
        l2.append([hi, lo])
    l4 = [_merge22(l2[2 * j], l2[2 * j + 1]) for j in range(len(l2) // 2)]
    l8 = [_oem44(l4[2 * j], l4[2 * j + 1]) for j in range(len(l4) // 2)]

    p = _merge_top6(_merge_top6(l8[0], l8[1]), _merge_top6(l8[2], l8[3]))
    a8 = [x[::-1] for x in l8]
    nlist = _merge_bot6(_merge_bot6(a8[0], a8[1]), _merge_bot6(a8[2], a8[3]))

    bf = jnp.bfloat16
    lane = jax.lax.broadcasted_iota(
        jnp.int32, sim.shape[:1] + (_CHUNK,), 1).astype(bf)
    big = bf(1024.0)

    plist = p[:_SIGMA + 1]
    s_rest = jnp.zeros(sim.shape[:1] + (1,), jnp.float32)
    for k in range(_SIGMA + 1):
        m = jnp.max(plist[0], axis=1, keepdims=True)
        if k > 0:
            s_rest = s_rest + m.astype(jnp.float32)
        if len(plist) > 1:
            mask = _first_lane_mask(plist[0] == m, lane, big)
            plist = [jnp.where(mask, plist[j + 1], plist[j])
                     for j in range(len(plist) - 1)]

    nl = nlist[:_SIGMA]
    mins = []
    for _ in range(_SIGMA):
        m = jnp.min(nl[0], axis=1, keepdims=True)
        mins.append(m.astype(jnp.float32))
        if len(nl) > 1:
            mask = _first_lane_mask(nl[0] == m, lane, big)
            nl = [jnp.where(mask, nl[j + 1], nl[j])
                  for j in range(len(nl) - 1)]

    return s_rest, jnp.concatenate(mins, axis=1)


def _loss_kernel(f_ref, out_ref, fn_ref, acc_ref):
    i = pl.program_id(0)
    n_steps = pl.num_programs(0)
    n_total = f_ref.shape[0]

    @pl.when(i == 0)
    def _norm():
        f = f_ref[...]
        nrm = jnp.sqrt(jnp.sum(f * f, axis=1, keepdims=True))
        fn_ref[...] = (f / jnp.maximum(nrm, 1e-12)).astype(jnp.bfloat16)
        acc_ref[0] = 0.0

    fn = fn_ref[...]
    total = jnp.float32(0.0)
    for s in range(_STRIPS):
        fb = fn_ref[pl.ds(i * (_STRIPS * _ROWS) + s * _ROWS, _ROWS), :]
        sim = jax.lax.dot_general(
            fb, fn, (((1,), (1,)), ((), ())),
            preferred_element_type=jnp.float32)
        s_rest, mins = _strip_v(sim)
        nsum = jnp.sum(jnp.exp(mins * (1.0 / _TEMP)), axis=1, keepdims=True)
        v = s_rest * (1.0 / (_SIGMA * _TEMP)) - jnp.log(nsum)
        total = total + jnp.sum(v)
    acc_ref[0] += total

    @pl.when(i == n_steps - 1)
    def _fin():
        out_ref[0] = jnp.maximum(
            jnp.float32(_MARGIN) - acc_ref[0] / n_total, jnp.float32(0.0))


def _build(n, d, interpret=False):
    return pl.pallas_call(
        _loss_kernel,
        grid=(n // (_ROWS * _STRIPS),),
        in_specs=[pl.BlockSpec((n, d), lambda i: (0, 0))],
        out_specs=pl.BlockSpec(memory_space=pltpu.SMEM),
        out_shape=jax.ShapeDtypeStruct((1,), jnp.float32),
        scratch_shapes=[
            pltpu.VMEM((n, d), jnp.bfloat16),
            pltpu.SMEM((1,), jnp.float32),
        ],
        compiler_params=pltpu.CompilerParams(
            dimension_semantics=("arbitrary",)),
        interpret=interpret,
    )


@jax.jit
def kernel(features):
    f = features.reshape(features.shape[0], -1)
    n, d = f.shape
    out = _build(n, d)(f)
    return out[0]


# speedup vs baseline: 1.0190x; 1.0190x over previous
import jax
import jax.numpy as jnp
from jax.experimental import pallas as pl
from jax.experimental.pallas import tpu as pltpu

_SIGMA = 5
_TEMP = 0.1
_MARGIN = 10.0
_ROWS = 1024
_STRIPS = 4
_CHUNK = 128


def _ce(a, b):
    return jnp.maximum(a, b), jnp.minimum(a, b)


def _merge22(x, y):
    p, q = _ce(x[0], y[0])
    r, s = _ce(x[1], y[1])
    m1, m2 = _ce(q, r)
    return [p, m1, m2, s]


def _oem44(a, b):
    e = _merge22([a[0], a[2]], [b[0], b[2]])
    o = _merge22([a[1], a[3]], [b[1], b[3]])
    h1, l1 = _ce(o[0], e[1])
    h2, l2 = _ce(o[1], e[2])
    h3, l3 = _ce(o[2], e[3])
    return [e[0], h1, l1, h2, l2, h3, l3, o[3]]


def _clean6(t, desc):
    t = list(t)
    for i in range(3):
        hi, lo = _ce(t[i], t[i + 3])
        t[i], t[i + 3] = (hi, lo) if desc else (lo, hi)
    for base in (0, 3):
        for (x, y) in ((0, 1), (1, 2), (0, 1)):
            hi, lo = _ce(t[base + x], t[base + y])
            t[base + x], t[base + y] = (hi, lo) if desc else (lo, hi)
    return t


def _merge_top6(a, b):
    t = [jnp.maximum(a[i], b[5 - i]) for i in range(6)]
    return _clean6(t, desc=True)


def _merge_bot6(a, b):
    t = [jnp.minimum(a[i], b[5 - i]) for i in range(6)]
    return _clean6(t, desc=False)


def _first_lane_mask(eq, lane, big):
    first = jnp.min(jnp.where(eq, lane, big), axis=1, keepdims=True)
    return lane == first


def _strip_v(sim):
    n_chunks = sim.shape[1] // _CHUNK
    chunks = [sim[:, c * _CHUNK:(c + 1) * _CHUNK].astype(jnp.bfloat16)
              for c in range(n_chunks)]

    l2 = []
    for j in range(n_chunks // 2):
        hi, lo = _ce(chunks[2 * j], chunks[2 * j + 1])
        l2.append([hi, lo])
    l4 = [_merge22(l2[2 * j], l2[2 * j + 1]) for j in range(len(l2) // 2)]
    l8 = [_oem44(l4[2 * j], l4[2 * j + 1]) for j in range(len(l4) // 2)]

    p = _merge_top6(_merge_top6(l8[0], l8[1]), _merge_top6(l8[2], l8[3]))
    a8 = [x[::-1] for x in l8]
    nlist = _merge_bot6(_merge_bot6(a8[0], a8[1]), _merge_bot6(a8[2], a8[3]))

    bf = jnp.bfloat16
    lane = jax.lax.broadcasted_iota(
        jnp.int32, sim.shape[:1] + (_CHUNK,), 1).astype(bf)
    big = bf(1024.0)

    plist = p[:_SIGMA + 1]
    s_rest = jnp.zeros(sim.shape[:1] + (1,), jnp.float32)
    for k in range(_SIGMA + 1):
        m = jnp.max(plist[0], axis=1, keepdims=True)
        if k > 0:
            s_rest = s_rest + m.astype(jnp.float32)
        if len(plist) > 1:
            mask = _first_lane_mask(plist[0] == m, lane, big)
            plist = [jnp.where(mask, plist[j + 1], plist[j])
                     for j in range(len(plist) - 1)]

    nl = nlist[:_SIGMA]
    mins = []
    for _ in range(_SIGMA):
        m = jnp.min(nl[0], axis=1, keepdims=True)
        mins.append(m.astype(jnp.float32))
        if len(nl) > 1:
            mask = _first_lane_mask(nl[0] == m, lane, big)
            nl = [jnp.where(mask, nl[j + 1], nl[j])
                  for j in range(len(nl) - 1)]

    return s_rest, jnp.concatenate(mins, axis=1)


def _loss_kernel(f_ref, out_ref, fn_ref, acc_ref):
    i = pl.program_id(0)
    n_steps = pl.num_programs(0)
    n_total = f_ref.shape[0]

    @pl.when(i == 0)
    def _norm():
        f = f_ref[...]
        nrm = jnp.sqrt(jnp.sum(f * f, axis=1, keepdims=True))
        fn_ref[...] = (f / jnp.maximum(nrm, 1e-12)).astype(jnp.bfloat16)
        acc_ref[0] = 0.0

    fn = fn_ref[...]
    total = jnp.float32(0.0)
    for s in range(_STRIPS):
        fb = fn_ref[pl.ds(i * (_STRIPS * _ROWS) + s * _ROWS, _ROWS), :]
        sim = jax.lax.dot_general(
            fb, fn, (((1,), (1,)), ((), ())),
            preferred_element_type=jnp.float32)
        s_rest, mins = _strip_v(sim)
        nsum = jnp.sum(jnp.exp(mins * (1.0 / _TEMP)), axis=1, keepdims=True)
        v = s_rest * (1.0 / (_SIGMA * _TEMP)) - jnp.log(nsum)
        total = total + jnp.sum(v)
    acc_ref[0] += total

    @pl.when(i == n_steps - 1)
    def _fin():
        out_ref[0] = jnp.maximum(
            jnp.float32(_MARGIN) - acc_ref[0] / n_total, jnp.float32(0.0))


def _build(n, d, interpret=False):
    return pl.pallas_call(
        _loss_kernel,
        grid=(n // (_ROWS * _STRIPS),),
        in_specs=[pl.BlockSpec((n, d), lambda i: (0, 0))],
        out_specs=pl.BlockSpec(memory_space=pltpu.SMEM),
        out_shape=jax.ShapeDtypeStruct((1,), jnp.float32),
        scratch_shapes=[
            pltpu.VMEM((n, d), jnp.bfloat16),
            pltpu.SMEM((1,), jnp.float32),
        ],
        compiler_params=pltpu.CompilerParams(
            dimension_semantics=("arbitrary",)),
        interpret=interpret,
    )


@jax.jit
def kernel(features):
    f = features.reshape(features.shape[0], -1)
    n, d = f.shape
    out = _build(n, d)(f)
    return out[0]
